# baseline (speedup 1.0000x reference)
import jax
import jax.numpy as jnp
from jax import lax
from jax.experimental import pallas as pl
from jax.experimental.pallas import tpu as pltpu

N_BINS = 15
NPT = 100000
NCLS = 1000
BR = 1000
NBLK = NPT // BR
NQ = N_BINS + 1
LOG2E = 1.4426950408889634
LO_BITS = 0x3A800000
ONE_BITS = 0x3F800000
N_ITERS = 27
LW = 12544
PADN = 8 * LW - NPT
PAD_BITS = 0x40000000


def _rowstats_kernel(x_ref, lab_ref, conf_ref, acc_ref):
    x = x_ref[...]
    m = jnp.max(x, axis=1, keepdims=True)
    col = lax.broadcasted_iota(jnp.int32, (1, NCLS), 1)
    lab = lab_ref[0]
    at_lab = jnp.where(col == lab, x, -jnp.inf)
    v_at_label = jnp.max(at_lab, axis=1, keepdims=True)
    s = jnp.sum(jnp.exp2(x * LOG2E - m * LOG2E), axis=1, keepdims=True)
    conf_ref[0] = 1.0 / s
    acc_ref[0] = (v_at_label == m).astype(jnp.float32)


def _ece_kernel(conf_ref, acc_ref, pos_ref, ilo_ref, ece_ref):
    ilo = ilo_ref[...]
    needs = [ilo[0, r] + 1 for r in range(NQ)]

    def body(_, carry):
        los, his = carry
        mids = tuple((l + h) // 2 for l, h in zip(los, his))
        d = lax.bitcast_convert_type(conf_ref[...], jnp.int32)
        cnts = [jnp.sum((d <= mm).astype(jnp.int32)) for mm in mids]
        ge = [c >= n for c, n in zip(cnts, needs)]
        nlo = tuple(jnp.where(g, l, m + 1)
                    for g, l, m in zip(ge, los, mids))
        nhi = tuple(jnp.where(g, m, h)
                    for g, m, h in zip(ge, mids, his))
        return nlo, nhi

    los0 = tuple(jnp.int32(LO_BITS) for _ in range(NQ))
    his0 = tuple(jnp.int32(ONE_BITS) for _ in range(NQ))
    _, bitsA = lax.fori_loop(0, N_ITERS, body, (los0, his0))

    d = lax.bitcast_convert_type(conf_ref[...], jnp.int32)
    cntA = [jnp.sum((d <= bA).astype(jnp.int32)) for bA in bitsA]
    mngA = [jnp.min(jnp.where(d > bA, d, jnp.int32(PAD_BITS)))
            for bA in bitsA]
    bitsB = tuple(jnp.where(c >= n + 1, bA, mg)
                  for c, n, bA, mg in zip(cntA, needs, bitsA, mngA))

    qA = jnp.stack(bitsA).reshape(1, NQ)
    qB = jnp.stack(bitsB).reshape(1, NQ)
    s_lo = lax.bitcast_convert_type(qA, jnp.float32)
    s_hi = lax.bitcast_convert_type(qB, jnp.float32)

    pos = pos_ref[...]
    delta = pos - ilo.astype(jnp.float32)
    bvals = s_lo + delta * (s_hi - s_lo)
    bvals = jnp.where(pos > float(NPT - 1), s_hi, bvals)
    bv = [bvals[0, j] for j in range(NQ)]

    conf = conf_ref[...]
    acc = acc_ref[...]
    ccnt = [jnp.sum((conf <= b).astype(jnp.int32)).astype(jnp.float32)
            for b in bv]
    cacc = [jnp.sum(jnp.where(conf <= b, acc, 0.0)) for b in bv]
    cconf = [jnp.sum(jnp.where(conf <= b, conf, 0.0)) for b in bv]

    ece = jnp.float32(0.0)
    for b in range(N_BINS):
        cnt = ccnt[b + 1] - ccnt[b]
        sa = cacc[b + 1] - cacc[b]
        sc = cconf[b + 1] - cconf[b]
        prop = cnt / float(NPT)
        safe = jnp.maximum(cnt, 1.0)
        term = jnp.abs(sc / safe - sa / safe) * prop
        ece = ece + jnp.where(prop > 0, term, 0.0)
    ece_ref[...] = jnp.reshape(ece, (1, 1))


def kernel(logits, labels):
    labels3 = labels.reshape(NBLK, BR, 1)
    conf3, acc3 = pl.pallas_call(
        _rowstats_kernel,
        grid=(NBLK,),
        in_specs=[
            pl.BlockSpec((BR, NCLS), lambda i: (i, 0)),
            pl.BlockSpec((1, BR, 1), lambda i: (i, 0, 0)),
        ],
        out_specs=[
            pl.BlockSpec((1, BR, 1), lambda i: (i, 0, 0)),
            pl.BlockSpec((1, BR, 1), lambda i: (i, 0, 0)),
        ],
        out_shape=[
            jax.ShapeDtypeStruct((NBLK, BR, 1), jnp.float32),
            jax.ShapeDtypeStruct((NBLK, BR, 1), jnp.float32),
        ],
    )(logits, labels3)

    confp = jnp.concatenate(
        [conf3.reshape(-1), jnp.full((PADN,), 2.0, jnp.float32)]
    ).reshape(8, LW)
    accp = jnp.concatenate(
        [acc3.reshape(-1), jnp.zeros((PADN,), jnp.float32)]
    ).reshape(8, LW)

    pos = jnp.linspace(0.0, float(NPT), N_BINS + 1).reshape(1, N_BINS + 1)
    ilo = jnp.clip(jnp.floor(pos).astype(jnp.int32), 0, NPT - 2)

    ece = pl.pallas_call(
        _ece_kernel,
        out_shape=jax.ShapeDtypeStruct((1, 1), jnp.float32),
    )(confp, accp, pos, ilo)
    return ece.reshape(1)
